# baseline (speedup 1.0000x reference)
import functools

import jax
import jax.numpy as jnp
from jax import lax
from jax.experimental import pallas as pl
from jax.experimental.pallas import tpu as pltpu
from jax.experimental.pallas import tpu_sc as plsc

N_NODES = 10000
N_EDGES = 320000
IN_FEAT = 128
OUT_FEAT = 128
NUM_RELS = 64
NUM_BASES = 8
SI = IN_FEAT // NUM_BASES
SO = OUT_FEAT // NUM_BASES

NC = 2
NS = 16
NW = NC * NS

BLK_E = 1024
NBLK = (N_EDGES + BLK_E - 1) // BLK_E
E_PAD = NBLK * BLK_E
ROWS_T = 320
GROWS = 64
MNBUF = 4
GNBUF = 4
C_CAP = 3072
DRAIN_T = C_CAP - BLK_E
LIST_SZ = C_CAP + 80
NG8 = (NBLK + 7) // 8
OBLK = NG8 * 8


def _mm_body(h_ref, w_ref, o_ref):
    o_ref[0] = jnp.dot(h_ref[...], w_ref[0],
                       preferred_element_type=jnp.float32)


def _compute_table(h_bf, wbd_bf):
    return pl.pallas_call(
        _mm_body,
        grid=(NUM_RELS,),
        in_specs=[
            pl.BlockSpec((N_NODES, IN_FEAT), lambda r: (0, 0)),
            pl.BlockSpec((1, IN_FEAT, OUT_FEAT), lambda r: (r, 0, 0)),
        ],
        out_specs=pl.BlockSpec((1, N_NODES, OUT_FEAT), lambda r: (r, 0, 0)),
        out_shape=jax.ShapeDtypeStruct((NUM_RELS, N_NODES, OUT_FEAT),
                                       jnp.float32),
    )(h_bf, wbd_bf)


def _sc_scatter(meta2, nrm2, occ_t, biasb, table):
    mesh = plsc.VectorSubcoreMesh(core_axis_name="c", subcore_axis_name="s",
                                  num_cores=NC, num_subcores=NS)

    @functools.partial(
        pl.kernel,
        out_type=jax.ShapeDtypeStruct((N_NODES, OUT_FEAT), jnp.float32),
        mesh=mesh,
        scratch_types=[
            pltpu.VMEM((MNBUF, 2, BLK_E), jnp.int32),
            pltpu.VMEM((MNBUF, BLK_E), jnp.float32),
            pltpu.VMEM((2, 8, 64), jnp.int32),
            pltpu.VMEM((LIST_SZ,), jnp.int32),
            pltpu.VMEM((LIST_SZ,), jnp.int32),
            pltpu.VMEM((LIST_SZ,), jnp.float32),
            pltpu.VMEM((GNBUF, GROWS, OUT_FEAT), jnp.float32),
            pltpu.VMEM((ROWS_T, OUT_FEAT), jnp.float32),
            pltpu.SemaphoreType.DMA((MNBUF,)),
            pltpu.SemaphoreType.DMA((MNBUF,)),
            pltpu.SemaphoreType.DMA((2,)),
            pltpu.SemaphoreType.DMA((GNBUF,)),
        ],
    )
    def k(meta_hbm, nrm_hbm, occ_hbm, bias_hbm, t_hbm, out_hbm,
          meta_v, nrmf_v, occ_v, idx_l, dst_l, nrm_l, msg_v, acc,
          msem, nsem, osem, gsem):
        c = lax.axis_index("c")
        s = lax.axis_index("s")
        wid = s * NC + c
        lo = wid * ROWS_T
        row0 = pl.multiple_of(wid * ROWS_T, 8)

        pltpu.sync_copy(bias_hbm, acc)

        def _mload(blk, b):
            pltpu.async_copy(meta_hbm.at[blk], meta_v.at[b], msem.at[b])
            pltpu.async_copy(nrm_hbm.at[blk], nrmf_v.at[b], nsem.at[b])

        def _oload(g8, ob):
            g0 = pl.multiple_of(g8 * 8, 8)
            return pltpu.async_copy(occ_hbm.at[wid, pl.ds(g0, 8)],
                                    occ_v.at[ob], osem.at[ob])

        def _gather(cc, b):
            return pltpu.async_copy(
                t_hbm.at[idx_l.at[pl.ds(cc * GROWS, GROWS)]],
                msg_v.at[b], gsem.at[b])

        def _drain(n):
            zz = jnp.zeros((16,), jnp.int32)
            zf = jnp.zeros((16,), jnp.float32)
            for kk in range(GROWS // 16):
                sl = pl.ds(n + kk * 16, 16)
                idx_l[sl] = zz
                dst_l[sl] = zz
                nrm_l[sl] = zf
            nch = lax.div(n + GROWS - 1, GROWS)

            for p in range(GNBUF - 1):
                @pl.when(p < nch)
                def _():
                    _gather(p, p)

            def _step(cc, cy):
                b = lax.rem(cc, GNBUF)
                pltpu.make_async_copy(
                    t_hbm.at[idx_l.at[pl.ds(cc * GROWS, GROWS)]],
                    msg_v.at[b], gsem.at[b]).wait()

                def _grp(gg, cy2):
                    base = cc * GROWS + gg * 16
                    dlv = dst_l[pl.ds(base, 16)]
                    nv = nrm_l[pl.ds(base, 16)]
                    for l in range(16):
                        dr = dlv[l]
                        nsc = nv[l]
                        e = gg * 16 + l
                        for bb in range(OUT_FEAT // 16):
                            slf = pl.ds(bb * 16, 16)
                            plsc.addupdate(acc.at[dr, slf],
                                           msg_v[b, e, slf] * nsc)
                    return cy2
                lax.fori_loop(0, GROWS // 16, _grp, 0)

                @pl.when(cc + GNBUF - 1 < nch)
                def _():
                    _gather(cc + GNBUF - 1, lax.rem(cc + GNBUF - 1, GNBUF))
                return cy
            lax.fori_loop(0, nch, _step, 0)

        for p in range(MNBUF - 1):
            _mload(p, p)
        _oload(0, 0)
        _oload(1, 1)

        def _block(blk, ptr):
            b = lax.rem(blk, MNBUF)
            pltpu.make_async_copy(meta_hbm.at[blk], meta_v.at[b],
                                  msem.at[b]).wait()
            pltpu.make_async_copy(nrm_hbm.at[blk], nrmf_v.at[b],
                                  nsem.at[b]).wait()

            @pl.when(blk + MNBUF - 1 < NBLK)
            def _():
                _mload(blk + MNBUF - 1, lax.rem(blk + MNBUF - 1, MNBUF))

            g8 = lax.div(blk, 8)
            r8 = lax.rem(blk, 8)
            ob = lax.rem(g8, 2)

            @pl.when(r8 == 0)
            def _():
                g0 = pl.multiple_of(g8 * 8, 8)
                pltpu.make_async_copy(occ_hbm.at[wid, pl.ds(g0, 8)],
                                      occ_v.at[ob], osem.at[ob]).wait()

                @pl.when((blk > 0) & (g8 + 1 < NG8))
                def _():
                    _oload(g8 + 1, 1 - ob)

            def _q(q, p2):
                ov = occ_v[ob, r8, pl.ds(q * 16, 16)]
                for li in range(16):
                    mk = ov[li]
                    sl = pl.ds((q * 16 + li) * 16, 16)

                    def _app(p2i, mk=mk, sl=sl):
                        iv = meta_v[b, 0, sl]
                        dl = meta_v[b, 1, sl] - lo
                        nv = nrmf_v[b, sl]

                        def _half(mh, l0, p2h):
                            def _do(p2j):
                                for l in range(l0, l0 + 8):
                                    osl = pl.ds(p2j, 16)
                                    idx_l[osl] = jnp.full((16,), iv[l],
                                                          jnp.int32)
                                    dst_l[osl] = jnp.full((16,), dl[l],
                                                          jnp.int32)
                                    nrm_l[osl] = jnp.full((16,), nv[l],
                                                          jnp.float32)
                                    p2j = p2j + ((mh >> (l - l0)) & 1)
                                return p2j
                            return lax.cond(mh != 0, _do, lambda x: x, p2h)

                        p2i = _half(mk & 0xFF, 0, p2i)
                        return _half((mk >> 8) & 0xFF, 8, p2i)
                    p2 = lax.cond(mk != 0, _app, lambda x: x, p2)
                return p2
            ptr = lax.fori_loop(0, 4, _q, ptr)

            def _dr(p2):
                _drain(p2)
                return jnp.int32(0)
            return lax.cond(ptr >= DRAIN_T, _dr, lambda p2: p2, ptr)
        ptr = lax.fori_loop(0, NBLK, _block, jnp.int32(0))
        _drain(ptr)

        @pl.when(wid < NW - 1)
        def _():
            pltpu.sync_copy(acc.at[pl.ds(0, ROWS_T)],
                            out_hbm.at[pl.ds(row0, ROWS_T)])

        @pl.when(wid == NW - 1)
        def _():
            pltpu.sync_copy(acc.at[pl.ds(0, N_NODES - (NW - 1) * ROWS_T)],
                            out_hbm.at[pl.ds(row0,
                                             N_NODES - (NW - 1) * ROWS_T)])

    return k(meta2, nrm2, occ_t, biasb, table)


def kernel(h, edge_index, etype, norm, weight, h_bias):
    h = h.astype(jnp.float32)
    src = edge_index[0].astype(jnp.int32)
    dst = edge_index[1].astype(jnp.int32)
    ety = etype.astype(jnp.int32)
    nrm = norm.reshape(-1).astype(jnp.float32)

    w4 = weight.reshape(NUM_RELS, NUM_BASES, SI, SO)
    wbd = jnp.zeros((NUM_RELS, NUM_BASES, SI, NUM_BASES, SO), weight.dtype)
    for b in range(NUM_BASES):
        wbd = wbd.at[:, b, :, b, :].set(w4[:, b])
    wbd = wbd.reshape(NUM_RELS, IN_FEAT, OUT_FEAT)

    t = _compute_table(h.astype(jnp.bfloat16), wbd.astype(jnp.bfloat16))
    t2 = t.reshape(NUM_RELS * N_NODES, OUT_FEAT)

    padn = ((0, E_PAD - N_EDGES),)
    idxh = jnp.pad(ety * N_NODES + src, padn)
    dstp = jnp.pad(dst, padn, constant_values=-1)
    meta2 = jnp.stack([idxh.reshape(NBLK, BLK_E),
                       dstp.reshape(NBLK, BLK_E)], axis=1)
    nrm2 = jnp.pad(nrm, padn).reshape(NBLK, BLK_E)

    nvec = E_PAD // 16
    to16 = (dstp // ROWS_T).reshape(nvec, 16)
    lane_bit = (jnp.int32(1) << (jnp.arange(16, dtype=jnp.int32)))
    onehot = (to16[:, :, None] ==
              jnp.arange(NW, dtype=jnp.int32)[None, None, :])
    occv = (onehot * lane_bit[None, :, None]).sum(axis=1,
                                                  dtype=jnp.int32)
    occ_t = jnp.pad(occv.T.reshape(NW, NBLK, 64),
                    ((0, 0), (0, OBLK - NBLK), (0, 0)))

    biasb = jnp.tile(h_bias.reshape(1, OUT_FEAT), (ROWS_T, 1))
    return _sc_scatter(meta2, nrm2, occ_t, biasb, t2)


# speedup vs baseline: 1.1793x; 1.1793x over previous
import functools

import jax
import jax.numpy as jnp
from jax import lax
from jax.experimental import pallas as pl
from jax.experimental.pallas import tpu as pltpu
from jax.experimental.pallas import tpu_sc as plsc

N_NODES = 10000
N_EDGES = 320000
IN_FEAT = 128
OUT_FEAT = 128
NUM_RELS = 64
NUM_BASES = 8
SI = IN_FEAT // NUM_BASES
SO = OUT_FEAT // NUM_BASES

NC = 2
NS = 16
NW = NC * NS

EDGES_PER_TILE = N_EDGES // NW
CHUNK = 64
SG = 16
N_GROUPS = 10
EDGES_PAD = N_GROUPS * SG * CHUNK
ROWS_BIG = 632
ROWS_SMALL = 624

MM_BLK = 400
COMB_BLK = 2000


def _mm_body(h_ref, w_ref, o_ref):
    o_ref[0] = jnp.dot(h_ref[...], w_ref[0],
                       preferred_element_type=jnp.float32)


def _compute_table(h_bf, wbd_bf):
    return pl.pallas_call(
        _mm_body,
        grid=(NUM_RELS,),
        in_specs=[
            pl.BlockSpec((N_NODES, IN_FEAT), lambda r: (0, 0)),
            pl.BlockSpec((1, IN_FEAT, OUT_FEAT), lambda r: (r, 0, 0)),
        ],
        out_specs=pl.BlockSpec((1, N_NODES, OUT_FEAT), lambda r: (r, 0, 0)),
        out_shape=jax.ShapeDtypeStruct((NUM_RELS, N_NODES, OUT_FEAT),
                                       jnp.float32),
    )(h_bf, wbd_bf)


def _sc_scatter(src3, ety3, dst3, nrm3, table, zeros):
    mesh = plsc.VectorSubcoreMesh(core_axis_name="c", subcore_axis_name="s",
                                  num_cores=NC, num_subcores=NS)

    @functools.partial(
        pl.kernel,
        out_type=jax.ShapeDtypeStruct((NC, N_NODES, OUT_FEAT), jnp.float32),
        mesh=mesh,
        scratch_types=[
            pltpu.VMEM((SG, CHUNK), jnp.int32),
            pltpu.VMEM((SG, CHUNK), jnp.int32),
            pltpu.VMEM((SG, CHUNK), jnp.int32),
            pltpu.VMEM((SG, CHUNK), jnp.float32),
            pltpu.VMEM((2, CHUNK, OUT_FEAT), jnp.float32),
            pltpu.VMEM_SHARED((N_NODES, OUT_FEAT), jnp.float32),
            pltpu.SemaphoreType.DMA((2,)),
            pltpu.SemaphoreType.DMA((2,)),
        ],
    )
    def k(src_hbm, ety_hbm, dst_hbm, nrm_hbm, t_hbm, zeros_hbm, out_hbm,
          src_v, idx_v, dst_v, nrm_v, msg_v, acc_sh, gsem, ssem):
        c = lax.axis_index("c")
        s = lax.axis_index("s")
        wid = s * NC + c
        row0 = pl.multiple_of(
            s * ROWS_SMALL + 8 * jnp.minimum(s, 2), 8)

        @pl.when(s < 2)
        def _():
            pltpu.sync_copy(zeros_hbm.at[pl.ds(row0, ROWS_BIG)],
                            acc_sh.at[pl.ds(row0, ROWS_BIG)])

        @pl.when(s >= 2)
        def _():
            pltpu.sync_copy(zeros_hbm.at[pl.ds(row0, ROWS_SMALL)],
                            acc_sh.at[pl.ds(row0, ROWS_SMALL)])

        plsc.subcore_barrier()

        def _gather(cc, b):
            return pltpu.async_copy(t_hbm.at[idx_v.at[cc]], msg_v.at[b],
                                    gsem.at[b])

        def _wait_scatter(cc, b):
            pltpu.make_async_copy(msg_v.at[b], acc_sh.at[dst_v.at[cc]],
                                  ssem.at[b]).wait()

        def _group(g, cy):
            c0 = pl.multiple_of(g * SG, SG)
            pltpu.sync_copy(src_hbm.at[wid, pl.ds(c0, SG)], src_v)
            pltpu.sync_copy(ety_hbm.at[wid, pl.ds(c0, SG)], idx_v)
            pltpu.sync_copy(dst_hbm.at[wid, pl.ds(c0, SG)], dst_v)
            pltpu.sync_copy(nrm_hbm.at[wid, pl.ds(c0, SG)], nrm_v)

            def _cidx(j, cy2):
                for v in range(CHUNK // 16):
                    sl = pl.ds(v * 16, 16)
                    idx_v[j, sl] = idx_v[j, sl] * N_NODES + src_v[j, sl]
                return cy2
            lax.fori_loop(0, SG, _cidx, 0)

            _gather(0, 0)

            def _step(cc, cy2):
                b = lax.rem(cc, 2)
                pltpu.make_async_copy(t_hbm.at[idx_v.at[cc]], msg_v.at[b],
                                      gsem.at[b]).wait()

                def _scale(gg, cy3):
                    nv = nrm_v[cc, pl.ds(gg * 16, 16)]
                    for l in range(16):
                        nsc = nv[l]
                        e = gg * 16 + l
                        for bb in range(OUT_FEAT // 16):
                            sl = pl.ds(bb * 16, 16)
                            msg_v[b, e, sl] = msg_v[b, e, sl] * nsc
                    return cy3
                lax.fori_loop(0, CHUNK // 16, _scale, 0)
                pltpu.async_copy(msg_v.at[b], acc_sh.at[dst_v.at[cc]],
                                 ssem.at[b], add=True)

                @pl.when(cc <= SG - 2)
                def _():
                    @pl.when(cc >= 1)
                    def _():
                        _wait_scatter(cc - 1, 1 - b)
                    _gather(cc + 1, 1 - b)
                return cy2
            lax.fori_loop(0, SG, _step, 0)

            _wait_scatter(SG - 2, lax.rem(SG - 2, 2))
            _wait_scatter(SG - 1, lax.rem(SG - 1, 2))
            return cy
        lax.fori_loop(0, N_GROUPS, _group, 0)
        plsc.subcore_barrier()

        @pl.when(s < 2)
        def _():
            pltpu.sync_copy(acc_sh.at[pl.ds(row0, ROWS_BIG)],
                            out_hbm.at[c, pl.ds(row0, ROWS_BIG)])

        @pl.when(s >= 2)
        def _():
            pltpu.sync_copy(acc_sh.at[pl.ds(row0, ROWS_SMALL)],
                            out_hbm.at[c, pl.ds(row0, ROWS_SMALL)])

    return k(src3, ety3, dst3, nrm3, table, zeros)


def _comb_body(p_ref, b_ref, o_ref):
    o_ref[...] = p_ref[0] + p_ref[1] + b_ref[...]


def _combine(partial, bias2d):
    return pl.pallas_call(
        _comb_body,
        grid=(N_NODES // COMB_BLK,),
        in_specs=[
            pl.BlockSpec((NC, COMB_BLK, OUT_FEAT), lambda i: (0, i, 0)),
            pl.BlockSpec((1, OUT_FEAT), lambda i: (0, 0)),
        ],
        out_specs=pl.BlockSpec((COMB_BLK, OUT_FEAT), lambda i: (i, 0)),
        out_shape=jax.ShapeDtypeStruct((N_NODES, OUT_FEAT), jnp.float32),
    )(partial, bias2d)


def kernel(h, edge_index, etype, norm, weight, h_bias):
    h = h.astype(jnp.float32)
    src = edge_index[0].astype(jnp.int32)
    dst = edge_index[1].astype(jnp.int32)
    ety = etype.astype(jnp.int32)
    nrm = norm.reshape(-1).astype(jnp.float32)

    w4 = weight.reshape(NUM_RELS, NUM_BASES, SI, SO)
    wbd = jnp.zeros((NUM_RELS, NUM_BASES, SI, NUM_BASES, SO), weight.dtype)
    for b in range(NUM_BASES):
        wbd = wbd.at[:, b, :, b, :].set(w4[:, b])
    wbd = wbd.reshape(NUM_RELS, IN_FEAT, OUT_FEAT)

    t = _compute_table(h.astype(jnp.bfloat16), wbd.astype(jnp.bfloat16))
    t2 = t.reshape(NUM_RELS * N_NODES, OUT_FEAT)
    zeros = jnp.zeros((N_NODES, OUT_FEAT), jnp.float32)

    pad = ((0, 0), (0, EDGES_PAD - EDGES_PER_TILE))
    shp = (NW, N_GROUPS * SG, CHUNK)
    srcp = jnp.pad(src.reshape(NW, EDGES_PER_TILE), pad).reshape(shp)
    etyp = jnp.pad(ety.reshape(NW, EDGES_PER_TILE), pad).reshape(shp)
    dstp = jnp.pad(dst.reshape(NW, EDGES_PER_TILE), pad).reshape(shp)
    nrmp = jnp.pad(nrm.reshape(NW, EDGES_PER_TILE), pad).reshape(shp)
    part = _sc_scatter(srcp, etyp, dstp, nrmp, t2, zeros)
    return _combine(part, h_bias.reshape(1, OUT_FEAT))
